# baseline (speedup 1.0000x reference)
import jax
import jax.numpy as jnp
from jax import lax
from jax.experimental import pallas as pl
from jax.experimental.pallas import tpu as pltpu

_B, _N, _C = 8, 2048, 131
_G = 4
_NPT = 512


def _fps_kernel(x2d_ref, xt_ref, out_ref, corr0_ref, corr1_ref, asq0_ref,
                asq1_ref):
    iota = lax.broadcasted_iota(jnp.int32, (_G, _N), 1)
    iotag = lax.broadcasted_iota(jnp.int32, (_G, 1), 0)
    corr_refs = (corr0_ref, corr1_ref)
    asq_refs = (asq0_ref, asq1_ref)

    for b in range(_B):
        xb = xt_ref[b]
        asq_refs[b // _G][b % _G:b % _G + 1, :] = jnp.sum(
            xb * xb, axis=0, keepdims=True)
    a_sq = (asq0_ref[...], asq1_ref[...])

    def _scal(vec, b):
        return jnp.max(jnp.where(iotag == b, vec, -1))

    def _row_extract(mat, idx_vec, fill):
        return jnp.max(jnp.where(iota == idx_vec, mat, fill), axis=1,
                       keepdims=True)

    def body(i, carry):
        fs, sqf, dists = carry
        new_fs = [None] * _B
        new_sqf, new_dists = [], []
        for g in range(2):
            for b in range(_G):
                bb = g * _G + b
                out_ref[i, bb] = fs[bb]
                row = x2d_ref[pl.ds(bb * _N + fs[bb], 1), :]
                corr_refs[g][b:b + 1, :] = lax.dot_general(
                    row, xt_ref[bb],
                    dimension_numbers=(((1,), (0,)), ((), ())),
                    preferred_element_type=jnp.float32,
                )
        for g in range(2):
            corr = corr_refs[g][...]
            d = (sqf[g] + a_sq[g]) - 2.0 * corr
            nd = jnp.minimum(dists[g], d)
            m = jnp.max(nd, axis=1, keepdims=True)
            nf = jnp.min(jnp.where(nd == m, iota, _N), axis=1,
                         keepdims=True).astype(jnp.int32)
            new_sqf.append(_row_extract(a_sq[g], nf, -jnp.inf))
            new_dists.append(nd)
            for b in range(_G):
                new_fs[g * _G + b] = _scal(nf, b)
        return tuple(new_fs), tuple(new_sqf), tuple(new_dists)

    fs0 = tuple(jnp.int32(0) for _ in range(_B))
    z = jnp.zeros((_G, 1), jnp.int32)
    sqf0 = tuple(_row_extract(a_sq[g], z, -jnp.inf) for g in range(2))
    dists0 = tuple(jnp.full((_G, _N), 1e10, jnp.float32) for _ in range(2))
    lax.fori_loop(0, _NPT, body, (fs0, sqf0, dists0))


def kernel(points_xyz, features):
    feats_t = jnp.transpose(features, (0, 2, 1))
    xcat = jnp.concatenate([points_xyz, feats_t], axis=2)
    x2d = xcat.reshape(_B * _N, _C)
    xt = jnp.concatenate(
        [jnp.transpose(points_xyz, (0, 2, 1)), features], axis=1)
    out = pl.pallas_call(
        _fps_kernel,
        out_shape=jax.ShapeDtypeStruct((_NPT, _B), jnp.int32),
        in_specs=[
            pl.BlockSpec(memory_space=pltpu.VMEM),
            pl.BlockSpec(memory_space=pltpu.VMEM),
        ],
        out_specs=pl.BlockSpec(memory_space=pltpu.SMEM),
        scratch_shapes=[
            pltpu.VMEM((_G, _N), jnp.float32),
            pltpu.VMEM((_G, _N), jnp.float32),
            pltpu.VMEM((_G, _N), jnp.float32),
            pltpu.VMEM((_G, _N), jnp.float32),
        ],
    )(x2d, xt)
    return jnp.transpose(out, (1, 0))
